# baseline (speedup 1.0000x reference)
import jax
import jax.numpy as jnp
from jax import lax
from jax.experimental import pallas as pl
from jax.experimental.pallas import tpu as pltpu
from jax.experimental.pallas import tpu_sc as plsc

NS, L, HD = 2, 2048, 1024
MLP = 4096
E = 8
GS = 512
T = NS * L

BB = 256
NT = T * 2 // BB + E - 1
NROWS = NT * BB
MLPC = 1024
J2 = MLP // MLPC

NW = 32
DCH = 64
DNC = T // (NW * DCH)
CCH = 32
CNC = T // (NW * CCH)


def _router_body(x_ref, wr_ref, br_ref, gates_ref, aux_ref, pos0_ref,
                 pos1_ref, c0_ref, c1_ref, te_ref):
    x = x_ref[...]
    logits = lax.dot_general(
        x, wr_ref[...], (((1,), (0,)), ((), ())),
        preferred_element_type=jnp.float32) + br_ref[...]
    m = jnp.max(logits, axis=1, keepdims=True)
    p = jnp.exp(logits - m)
    probs = p / jnp.sum(p, axis=1, keepdims=True)
    gates_ref[...] = probs

    eidx = lax.broadcasted_iota(jnp.int32, probs.shape, 1)
    v0 = jnp.max(probs, axis=1, keepdims=True)
    i0 = jnp.min(jnp.where(probs == v0, eidx, E), axis=1, keepdims=True)
    m0 = eidx == i0
    probs2 = jnp.where(m0, -jnp.inf, probs)
    v1 = jnp.max(probs2, axis=1, keepdims=True)
    i1 = jnp.min(jnp.where(probs2 == v1, eidx, E), axis=1, keepdims=True)
    m1 = eidx == i1
    s = v0 + v1
    w0 = v0 / (s + 1e-9)
    w1 = v1 / (s + 1e-9)
    d = w0 + w1 + 1e-9
    c0_ref[...] = jnp.broadcast_to(w0 / d, (T, 16))
    c1_ref[...] = jnp.broadcast_to(w1 / d, (T, 16))

    imp = jnp.sum(probs, axis=0, keepdims=True)
    load = jnp.sum((probs > 0).astype(jnp.float32), axis=0, keepdims=True)
    il = imp * load
    mu = jnp.mean(il)
    aux = jnp.sum((il - mu) ** 2) / (E - 1) * 0.01
    aux_ref[...] = jnp.broadcast_to(aux, (1, 1))

    a = m0.astype(jnp.float32) + m1.astype(jnp.float32)
    inc = a
    k = 1
    while k < T:
        shifted = jnp.concatenate(
            [jnp.zeros((k, E), jnp.float32), inc[:-k, :]], axis=0)
        inc = inc + shifted
        k *= 2
    ex = inc - a
    counts = inc[T - 1:T, :]
    ntiles = jnp.ceil(counts * (1.0 / BB))
    r = lax.broadcasted_iota(jnp.int32, (E, E), 0)
    c = lax.broadcasted_iota(jnp.int32, (E, E), 1)
    strict_lower = (r < c).astype(jnp.float32)
    tile_start = lax.dot_general(
        ntiles, strict_lower, (((1,), (0,)), ((), ())),
        preferred_element_type=jnp.float32)
    row_start = tile_start * BB
    dest = row_start + ex
    pos0_ref[...] = jnp.sum(jnp.where(m0, dest, 0.0), axis=1,
                            keepdims=True).astype(jnp.int32)
    pos1_ref[...] = jnp.sum(jnp.where(m1, dest, 0.0), axis=1,
                            keepdims=True).astype(jnp.int32)

    tend = tile_start + ntiles
    tt = lax.broadcasted_iota(jnp.int32, (NT, E), 0).astype(jnp.float32)
    full_before = jnp.sum((tend <= tt).astype(jnp.float32), axis=1,
                          keepdims=True)
    te_ref[...] = jnp.minimum(full_before, E - 1).astype(jnp.int32)


def _run_router(flat, Wr, br):
    return pl.pallas_call(
        _router_body,
        out_shape=[
            jax.ShapeDtypeStruct((T, E), jnp.float32),
            jax.ShapeDtypeStruct((1, 1), jnp.float32),
            jax.ShapeDtypeStruct((T, 1), jnp.int32),
            jax.ShapeDtypeStruct((T, 1), jnp.int32),
            jax.ShapeDtypeStruct((T, 16), jnp.float32),
            jax.ShapeDtypeStruct((T, 16), jnp.float32),
            jax.ShapeDtypeStruct((NT, 1), jnp.int32),
        ],
    )(flat, Wr, br.reshape(1, E))


def _dispatch_body(flat_hbm, pos0_hbm, pos1_hbm, xpad_hbm, idx_v, rows_v, sem):
    wid = lax.axis_index("s") * 2 + lax.axis_index("c")
    for slot in range(2):
        pos_hbm = pos0_hbm if slot == 0 else pos1_hbm
        for ch in range(DNC):
            base = wid * (DNC * DCH) + ch * DCH
            pltpu.sync_copy(pos_hbm.at[wid, ch], idx_v)
            pltpu.sync_copy(flat_hbm.at[pl.ds(base, DCH)], rows_v)
            pltpu.async_copy(rows_v, xpad_hbm.at[idx_v], sem).wait()


def _make_dispatch():
    return pl.kernel(
        _dispatch_body,
        out_type=jax.ShapeDtypeStruct((NROWS, HD), jnp.float32),
        mesh=plsc.VectorSubcoreMesh(core_axis_name="c", subcore_axis_name="s"),
        scratch_types=[
            pltpu.VMEM((DCH,), jnp.int32),
            pltpu.VMEM((DCH, HD), jnp.float32),
            pltpu.SemaphoreType.DMA,
        ],
    )


def _gmm_body(te_ref, x_ref, w1_ref, b1_ref, w2_ref, b2_ref, y_ref, h_ref):
    t = pl.program_id(0)
    j = pl.program_id(1)
    e = te_ref[t]

    @pl.when(j == 0)
    def _():
        b1 = b1_ref[pl.ds(e, 1), :]
        h = lax.dot_general(
            x_ref[...].astype(jnp.bfloat16), w1_ref[0],
            (((1,), (0,)), ((), ())),
            preferred_element_type=jnp.float32) + b1
        h_ref[...] = (h * 0.5 * (1.0 + lax.erf(h * 0.7071067811865476))
                      ).astype(jnp.bfloat16)
        y_ref[...] = jnp.broadcast_to(b2_ref[pl.ds(e, 1), :], y_ref.shape)

    y_ref[...] += lax.dot_general(
        h_ref[:, pl.ds(j * MLPC, MLPC)], w2_ref[0],
        (((1,), (0,)), ((), ())), preferred_element_type=jnp.float32)


def _run_gmm(te, xpad, W1, b1, W2, b2):
    grid_spec = pltpu.PrefetchScalarGridSpec(
        num_scalar_prefetch=1,
        grid=(NT, J2),
        in_specs=[
            pl.BlockSpec((BB, HD), lambda t, j, te: (t, 0)),
            pl.BlockSpec((1, HD, MLP), lambda t, j, te: (te[t], 0, 0)),
            pl.BlockSpec((E, MLP), lambda t, j, te: (0, 0)),
            pl.BlockSpec((1, MLPC, HD), lambda t, j, te: (te[t], j, 0)),
            pl.BlockSpec((E, HD), lambda t, j, te: (0, 0)),
        ],
        out_specs=pl.BlockSpec((BB, HD), lambda t, j, te: (t, 0)),
        scratch_shapes=[pltpu.VMEM((BB, MLP), jnp.bfloat16)],
    )
    return pl.pallas_call(
        _gmm_body,
        grid_spec=grid_spec,
        out_shape=jax.ShapeDtypeStruct((NROWS, HD), jnp.float32),
        compiler_params=pltpu.CompilerParams(
            dimension_semantics=("arbitrary", "arbitrary")),
    )(te, xpad, W1, b1, W2, b2)


def _combine_body(y_hbm, pos0_hbm, pos1_hbm, c0_hbm, c1_hbm, out_hbm,
                  idx0_v, idx1_v, r0_v, r1_v, cc0_v, cc1_v, sem):
    wid = lax.axis_index("s") * 2 + lax.axis_index("c")
    for ch in range(CNC):
        base = wid * (CNC * CCH) + ch * CCH
        pltpu.sync_copy(pos0_hbm.at[wid, ch], idx0_v)
        pltpu.sync_copy(pos1_hbm.at[wid, ch], idx1_v)
        pltpu.sync_copy(c0_hbm.at[wid, ch], cc0_v)
        pltpu.sync_copy(c1_hbm.at[wid, ch], cc1_v)
        pltpu.async_copy(y_hbm.at[idx0_v], r0_v, sem).wait()
        pltpu.async_copy(y_hbm.at[idx1_v], r1_v, sem).wait()

        def row(i, carry):
            a = cc0_v[i, :]
            b = cc1_v[i, :]

            def col(k, carry2):
                sl = pl.ds(k * 16, 16)
                r0_v[i, sl] = a * r0_v[i, sl] + b * r1_v[i, sl]
                return carry2

            return lax.fori_loop(0, HD // 16, col, carry)

        lax.fori_loop(0, CCH, row, 0)
        pltpu.sync_copy(r0_v, out_hbm.at[pl.ds(base, CCH)])


def _make_combine():
    return pl.kernel(
        _combine_body,
        out_type=jax.ShapeDtypeStruct((T, HD), jnp.float32),
        mesh=plsc.VectorSubcoreMesh(core_axis_name="c", subcore_axis_name="s"),
        scratch_types=[
            pltpu.VMEM((CCH,), jnp.int32),
            pltpu.VMEM((CCH,), jnp.int32),
            pltpu.VMEM((CCH, HD), jnp.float32),
            pltpu.VMEM((CCH, HD), jnp.float32),
            pltpu.VMEM((CCH, 16), jnp.float32),
            pltpu.VMEM((CCH, 16), jnp.float32),
            pltpu.SemaphoreType.DMA,
        ],
    )


def kernel(inputs, Wr, br, W1, b1, W2, b2):
    flat = inputs.reshape(T, HD)
    gates, aux, pos0, pos1, c0, c1, te = _run_router(flat, Wr, br)
    xpad = _make_dispatch()(flat,
                            pos0.reshape(NW, DNC, DCH),
                            pos1.reshape(NW, DNC, DCH))
    y = _run_gmm(te.reshape(NT), xpad,
                 W1.astype(jnp.bfloat16), b1, W2.astype(jnp.bfloat16), b2)
    out = _make_combine()(y,
                   pos0.reshape(NW, CNC, CCH),
                   pos1.reshape(NW, CNC, CCH),
                   c0.reshape(NW, CNC, CCH, 16),
                   c1.reshape(NW, CNC, CCH, 16))
    G = NS * (L // GS)
    return out.reshape(NS, L, HD), aux[0, 0], gates.reshape(G, GS, E)


# speedup vs baseline: 1.1124x; 1.1124x over previous
import jax
import jax.numpy as jnp
from jax import lax
from jax.experimental import pallas as pl
from jax.experimental.pallas import tpu as pltpu
from jax.experimental.pallas import tpu_sc as plsc

NS, L, HD = 2, 2048, 1024
MLP = 4096
E = 8
GS = 512
T = NS * L

BB = 256
NT = T * 2 // BB + E - 1
NROWS = NT * BB
MLPC = 1024
J2 = MLP // MLPC

NW = 32
DCH = 32
DNC = T // (NW * DCH)
DST = 2 * DNC
CCH = 16
CNC = T // (NW * CCH)


def _router_body(x_ref, wr_ref, br_ref, gates_ref, aux_ref, pos0_ref,
                 pos1_ref, c0_ref, c1_ref, te_ref):
    x = x_ref[...]
    logits = lax.dot_general(
        x, wr_ref[...], (((1,), (0,)), ((), ())),
        preferred_element_type=jnp.float32) + br_ref[...]
    m = jnp.max(logits, axis=1, keepdims=True)
    p = jnp.exp(logits - m)
    probs = p / jnp.sum(p, axis=1, keepdims=True)
    gates_ref[...] = probs

    eidx = lax.broadcasted_iota(jnp.int32, probs.shape, 1)
    v0 = jnp.max(probs, axis=1, keepdims=True)
    i0 = jnp.min(jnp.where(probs == v0, eidx, E), axis=1, keepdims=True)
    m0 = eidx == i0
    probs2 = jnp.where(m0, -jnp.inf, probs)
    v1 = jnp.max(probs2, axis=1, keepdims=True)
    i1 = jnp.min(jnp.where(probs2 == v1, eidx, E), axis=1, keepdims=True)
    m1 = eidx == i1
    s = v0 + v1
    w0 = v0 / (s + 1e-9)
    w1 = v1 / (s + 1e-9)
    d = w0 + w1 + 1e-9
    c0_ref[...] = jnp.broadcast_to(w0 / d, (T, 16))
    c1_ref[...] = jnp.broadcast_to(w1 / d, (T, 16))

    imp = jnp.sum(probs, axis=0, keepdims=True)
    load = jnp.sum((probs > 0).astype(jnp.float32), axis=0, keepdims=True)
    il = imp * load
    mu = jnp.mean(il)
    aux = jnp.sum((il - mu) ** 2) / (E - 1) * 0.01
    aux_ref[...] = jnp.broadcast_to(aux, (1, 1))

    a = m0.astype(jnp.float32) + m1.astype(jnp.float32)
    inc = a
    k = 1
    while k < T:
        shifted = jnp.concatenate(
            [jnp.zeros((k, E), jnp.float32), inc[:-k, :]], axis=0)
        inc = inc + shifted
        k *= 2
    ex = inc - a
    counts = inc[T - 1:T, :]
    ntiles = jnp.ceil(counts * (1.0 / BB))
    r = lax.broadcasted_iota(jnp.int32, (E, E), 0)
    c = lax.broadcasted_iota(jnp.int32, (E, E), 1)
    strict_lower = (r < c).astype(jnp.float32)
    tile_start = lax.dot_general(
        ntiles, strict_lower, (((1,), (0,)), ((), ())),
        preferred_element_type=jnp.float32)
    row_start = tile_start * BB
    dest = row_start + ex
    pos0_ref[...] = jnp.sum(jnp.where(m0, dest, 0.0), axis=1,
                            keepdims=True).astype(jnp.int32)
    pos1_ref[...] = jnp.sum(jnp.where(m1, dest, 0.0), axis=1,
                            keepdims=True).astype(jnp.int32)

    tend = tile_start + ntiles
    tt = lax.broadcasted_iota(jnp.int32, (NT, E), 0).astype(jnp.float32)
    full_before = jnp.sum((tend <= tt).astype(jnp.float32), axis=1,
                          keepdims=True)
    te_ref[...] = jnp.minimum(full_before, E - 1).astype(jnp.int32)


def _run_router(flat, Wr, br):
    return pl.pallas_call(
        _router_body,
        out_shape=[
            jax.ShapeDtypeStruct((T, E), jnp.float32),
            jax.ShapeDtypeStruct((1, 1), jnp.float32),
            jax.ShapeDtypeStruct((T, 1), jnp.int32),
            jax.ShapeDtypeStruct((T, 1), jnp.int32),
            jax.ShapeDtypeStruct((T, 16), jnp.float32),
            jax.ShapeDtypeStruct((T, 16), jnp.float32),
            jax.ShapeDtypeStruct((NT, 1), jnp.int32),
        ],
    )(flat, Wr, br.reshape(1, E))


def _dispatch_body(flat_hbm, pos0_hbm, pos1_hbm, xpad_hbm, pos_v, rows_v,
                   lsem, ssem):
    wid = lax.axis_index("s") * 2 + lax.axis_index("c")
    pltpu.sync_copy(pos0_hbm.at[wid], pos_v.at[pl.ds(0, DNC)])
    pltpu.sync_copy(pos1_hbm.at[wid], pos_v.at[pl.ds(DNC, DNC)])

    def tok_base(s):
        return wid * (DNC * DCH) + (s % DNC) * DCH

    loads = [None] * DST
    scats = [None] * DST
    loads[0] = pltpu.async_copy(
        flat_hbm.at[pl.ds(tok_base(0), DCH)], rows_v.at[0], lsem)
    for s in range(DST):
        b = s % 2
        loads[s].wait()
        if s + 1 < DST:
            if s >= 1:
                scats[s - 1].wait()
            loads[s + 1] = pltpu.async_copy(
                flat_hbm.at[pl.ds(tok_base(s + 1), DCH)],
                rows_v.at[1 - b], lsem)
        scats[s] = pltpu.async_copy(rows_v.at[b], xpad_hbm.at[pos_v.at[s]],
                                    ssem)
    scats[DST - 2].wait()
    scats[DST - 1].wait()


def _make_dispatch():
    return pl.kernel(
        _dispatch_body,
        out_type=jax.ShapeDtypeStruct((NROWS, HD), jnp.float32),
        mesh=plsc.VectorSubcoreMesh(core_axis_name="c", subcore_axis_name="s"),
        scratch_types=[
            pltpu.VMEM((DST, DCH), jnp.int32),
            pltpu.VMEM((2, DCH, HD), jnp.float32),
            pltpu.SemaphoreType.DMA,
            pltpu.SemaphoreType.DMA,
        ],
    )


def _gmm_body(te_ref, x_ref, w1_ref, b1_ref, w2_ref, b2_ref, y_ref, h_ref):
    t = pl.program_id(0)
    j = pl.program_id(1)
    e = te_ref[t]

    @pl.when(j == 0)
    def _():
        b1 = b1_ref[pl.ds(e, 1), :]
        h = lax.dot_general(
            x_ref[...], w1_ref[0], (((1,), (0,)), ((), ())),
            preferred_element_type=jnp.float32) + b1
        h_ref[...] = h * 0.5 * (1.0 + lax.erf(h * 0.7071067811865476))
        y_ref[...] = jnp.broadcast_to(b2_ref[pl.ds(e, 1), :], y_ref.shape)

    y_ref[...] += lax.dot_general(
        h_ref[:, pl.ds(j * MLPC, MLPC)], w2_ref[0],
        (((1,), (0,)), ((), ())),
        preferred_element_type=jnp.float32)


def _run_gmm(te, xpad, W1, b1, W2, b2):
    grid_spec = pltpu.PrefetchScalarGridSpec(
        num_scalar_prefetch=1,
        grid=(NT, J2),
        in_specs=[
            pl.BlockSpec((BB, HD), lambda t, j, te: (t, 0)),
            pl.BlockSpec((1, HD, MLP), lambda t, j, te: (te[t], 0, 0)),
            pl.BlockSpec((E, MLP), lambda t, j, te: (0, 0)),
            pl.BlockSpec((1, MLPC, HD), lambda t, j, te: (te[t], j, 0)),
            pl.BlockSpec((E, HD), lambda t, j, te: (0, 0)),
        ],
        out_specs=pl.BlockSpec((BB, HD), lambda t, j, te: (t, 0)),
        scratch_shapes=[pltpu.VMEM((BB, MLP), jnp.float32)],
    )
    return pl.pallas_call(
        _gmm_body,
        grid_spec=grid_spec,
        out_shape=jax.ShapeDtypeStruct((NROWS, HD), jnp.float32),
        compiler_params=pltpu.CompilerParams(
            dimension_semantics=("arbitrary", "arbitrary")),
    )(te, xpad, W1, b1, W2, b2)


def _combine_body(y_hbm, pos0_hbm, pos1_hbm, c0_hbm, c1_hbm, out_hbm,
                  idx0_v, idx1_v, r0_v, r1_v, cc0_v, cc1_v, gsem, ssem):
    wid = lax.axis_index("s") * 2 + lax.axis_index("c")
    pltpu.sync_copy(pos0_hbm.at[wid], idx0_v)
    pltpu.sync_copy(pos1_hbm.at[wid], idx1_v)
    pltpu.sync_copy(c0_hbm.at[wid], cc0_v)
    pltpu.sync_copy(c1_hbm.at[wid], cc1_v)

    g0 = [None] * CNC
    g1 = [None] * CNC
    stores = [None] * CNC
    g0[0] = pltpu.async_copy(y_hbm.at[idx0_v.at[0]], r0_v.at[0], gsem)
    g1[0] = pltpu.async_copy(y_hbm.at[idx1_v.at[0]], r1_v.at[0], gsem)
    for ch in range(CNC):
        b = ch % 2
        g0[ch].wait()
        g1[ch].wait()
        if ch + 1 < CNC:
            if ch >= 1:
                stores[ch - 1].wait()
            g0[ch + 1] = pltpu.async_copy(
                y_hbm.at[idx0_v.at[ch + 1]], r0_v.at[1 - b], gsem)
            g1[ch + 1] = pltpu.async_copy(
                y_hbm.at[idx1_v.at[ch + 1]], r1_v.at[1 - b], gsem)

        r0b = r0_v.at[b]
        r1b = r1_v.at[b]
        cc0c = cc0_v.at[ch]
        cc1c = cc1_v.at[ch]

        def row(i, carry):
            a = cc0c[i, :]
            bb = cc1c[i, :]

            def col(k, carry2):
                sl = pl.ds(k * 16, 16)
                r0b[i, sl] = a * r0b[i, sl] + bb * r1b[i, sl]
                return carry2

            return lax.fori_loop(0, HD // 16, col, carry)

        lax.fori_loop(0, CCH, row, 0)
        base = wid * (CNC * CCH) + ch * CCH
        stores[ch] = pltpu.async_copy(r0_v.at[b], out_hbm.at[pl.ds(base, CCH)],
                                      ssem)
    stores[CNC - 2].wait()
    stores[CNC - 1].wait()


def _make_combine():
    return pl.kernel(
        _combine_body,
        out_type=jax.ShapeDtypeStruct((T, HD), jnp.float32),
        mesh=plsc.VectorSubcoreMesh(core_axis_name="c", subcore_axis_name="s"),
        scratch_types=[
            pltpu.VMEM((CNC, CCH), jnp.int32),
            pltpu.VMEM((CNC, CCH), jnp.int32),
            pltpu.VMEM((2, CCH, HD), jnp.float32),
            pltpu.VMEM((2, CCH, HD), jnp.float32),
            pltpu.VMEM((CNC, CCH, 16), jnp.float32),
            pltpu.VMEM((CNC, CCH, 16), jnp.float32),
            pltpu.SemaphoreType.DMA,
            pltpu.SemaphoreType.DMA,
        ],
    )


def kernel(inputs, Wr, br, W1, b1, W2, b2):
    flat = inputs.reshape(T, HD)
    gates, aux, pos0, pos1, c0, c1, te = _run_router(flat, Wr, br)
    xpad = _make_dispatch()(flat,
                            pos0.reshape(NW, DNC, DCH),
                            pos1.reshape(NW, DNC, DCH))
    y = _run_gmm(te.reshape(NT), xpad, W1, b1, W2, b2)
    out = _make_combine()(y,
                   pos0.reshape(NW, CNC, CCH),
                   pos1.reshape(NW, CNC, CCH),
                   c0.reshape(NW, CNC, CCH, 16),
                   c1.reshape(NW, CNC, CCH, 16))
    G = NS * (L // GS)
    return out.reshape(NS, L, HD), aux[0, 0], gates.reshape(G, GS, E)


# speedup vs baseline: 1.2243x; 1.1007x over previous
import jax
import jax.numpy as jnp
from jax import lax
from jax.experimental import pallas as pl
from jax.experimental.pallas import tpu as pltpu
from jax.experimental.pallas import tpu_sc as plsc

NS, L, HD = 2, 2048, 1024
MLP = 4096
E = 8
GS = 512
T = NS * L

BB = 256
NT = T * 2 // BB + E - 1
NROWS = NT * BB
MLPC = 2048
J2 = MLP // MLPC

NW = 32
DCH = 32
DNC = T // (NW * DCH)
DST = 2 * DNC
CCH = 16
CNC = T // (NW * CCH)


def _router_body(x_ref, wr_ref, br_ref, gates_ref, aux_ref, pos0_ref,
                 pos1_ref, c0_ref, c1_ref, te_ref):
    x = x_ref[...]
    logits = lax.dot_general(
        x, wr_ref[...], (((1,), (0,)), ((), ())),
        preferred_element_type=jnp.float32) + br_ref[...]
    m = jnp.max(logits, axis=1, keepdims=True)
    p = jnp.exp(logits - m)
    probs = p / jnp.sum(p, axis=1, keepdims=True)
    gates_ref[...] = probs

    eidx = lax.broadcasted_iota(jnp.int32, probs.shape, 1)
    v0 = jnp.max(probs, axis=1, keepdims=True)
    i0 = jnp.min(jnp.where(probs == v0, eidx, E), axis=1, keepdims=True)
    m0 = eidx == i0
    probs2 = jnp.where(m0, -jnp.inf, probs)
    v1 = jnp.max(probs2, axis=1, keepdims=True)
    i1 = jnp.min(jnp.where(probs2 == v1, eidx, E), axis=1, keepdims=True)
    m1 = eidx == i1
    s = v0 + v1
    w0 = v0 / (s + 1e-9)
    w1 = v1 / (s + 1e-9)
    d = w0 + w1 + 1e-9
    c0_ref[...] = jnp.broadcast_to(w0 / d, (T, 16))
    c1_ref[...] = jnp.broadcast_to(w1 / d, (T, 16))

    imp = jnp.sum(probs, axis=0, keepdims=True)
    load = jnp.sum((probs > 0).astype(jnp.float32), axis=0, keepdims=True)
    il = imp * load
    mu = jnp.mean(il)
    aux = jnp.sum((il - mu) ** 2) / (E - 1) * 0.01
    aux_ref[...] = jnp.broadcast_to(aux, (1, 1))

    a = m0.astype(jnp.float32) + m1.astype(jnp.float32)
    inc = a
    k = 1
    while k < T:
        shifted = jnp.concatenate(
            [jnp.zeros((k, E), jnp.float32), inc[:-k, :]], axis=0)
        inc = inc + shifted
        k *= 2
    ex = inc - a
    counts = inc[T - 1:T, :]
    ntiles = jnp.ceil(counts * (1.0 / BB))
    r = lax.broadcasted_iota(jnp.int32, (E, E), 0)
    c = lax.broadcasted_iota(jnp.int32, (E, E), 1)
    strict_lower = (r < c).astype(jnp.float32)
    tile_start = lax.dot_general(
        ntiles, strict_lower, (((1,), (0,)), ((), ())),
        preferred_element_type=jnp.float32)
    row_start = tile_start * BB
    dest = row_start + ex
    pos0_ref[...] = jnp.sum(jnp.where(m0, dest, 0.0), axis=1,
                            keepdims=True).astype(jnp.int32)
    pos1_ref[...] = jnp.sum(jnp.where(m1, dest, 0.0), axis=1,
                            keepdims=True).astype(jnp.int32)

    tend = tile_start + ntiles
    tt = lax.broadcasted_iota(jnp.int32, (NT, E), 0).astype(jnp.float32)
    full_before = jnp.sum((tend <= tt).astype(jnp.float32), axis=1,
                          keepdims=True)
    te_ref[...] = jnp.minimum(full_before, E - 1).astype(jnp.int32)


def _run_router(flat, Wr, br):
    return pl.pallas_call(
        _router_body,
        out_shape=[
            jax.ShapeDtypeStruct((T, E), jnp.float32),
            jax.ShapeDtypeStruct((1, 1), jnp.float32),
            jax.ShapeDtypeStruct((T, 1), jnp.int32),
            jax.ShapeDtypeStruct((T, 1), jnp.int32),
            jax.ShapeDtypeStruct((T, 16), jnp.float32),
            jax.ShapeDtypeStruct((T, 16), jnp.float32),
            jax.ShapeDtypeStruct((NT, 1), jnp.int32),
        ],
    )(flat, Wr, br.reshape(1, E))


def _dispatch_body(flat_hbm, pos0_hbm, pos1_hbm, xpad_hbm, pos_v, rows_v,
                   lsem, ssem):
    wid = lax.axis_index("s") * 2 + lax.axis_index("c")
    pltpu.sync_copy(pos0_hbm.at[wid], pos_v.at[pl.ds(0, DNC)])
    pltpu.sync_copy(pos1_hbm.at[wid], pos_v.at[pl.ds(DNC, DNC)])

    def tok_base(s):
        return wid * (DNC * DCH) + (s % DNC) * DCH

    loads = [None] * DST
    scats = [None] * DST
    loads[0] = pltpu.async_copy(
        flat_hbm.at[pl.ds(tok_base(0), DCH)], rows_v.at[0], lsem)
    for s in range(DST):
        b = s % 2
        loads[s].wait()
        if s + 1 < DST:
            if s >= 1:
                scats[s - 1].wait()
            loads[s + 1] = pltpu.async_copy(
                flat_hbm.at[pl.ds(tok_base(s + 1), DCH)],
                rows_v.at[1 - b], lsem)
        scats[s] = pltpu.async_copy(rows_v.at[b], xpad_hbm.at[pos_v.at[s]],
                                    ssem)
    scats[DST - 2].wait()
    scats[DST - 1].wait()


def _make_dispatch():
    return pl.kernel(
        _dispatch_body,
        out_type=jax.ShapeDtypeStruct((NROWS, HD), jnp.float32),
        mesh=plsc.VectorSubcoreMesh(core_axis_name="c", subcore_axis_name="s"),
        scratch_types=[
            pltpu.VMEM((DST, DCH), jnp.int32),
            pltpu.VMEM((2, DCH, HD), jnp.float32),
            pltpu.SemaphoreType.DMA,
            pltpu.SemaphoreType.DMA,
        ],
    )


def _gmm_body(te_ref, x_ref, w1_ref, b1_ref, w2_ref, b2_ref, y_ref, h_ref):
    t = pl.program_id(0)
    j = pl.program_id(1)
    e = te_ref[t]

    @pl.when(j == 0)
    def _():
        b1 = b1_ref[pl.ds(e, 1), :]
        h = lax.dot_general(
            x_ref[...], w1_ref[0], (((1,), (0,)), ((), ())),
            preferred_element_type=jnp.float32) + b1
        h_ref[...] = h * 0.5 * (1.0 + lax.erf(h * 0.7071067811865476))
        y_ref[...] = jnp.broadcast_to(b2_ref[pl.ds(e, 1), :], y_ref.shape)

    y_ref[...] += lax.dot_general(
        h_ref[:, pl.ds(j * MLPC, MLPC)], w2_ref[0],
        (((1,), (0,)), ((), ())),
        preferred_element_type=jnp.float32)


def _run_gmm(te, xpad, W1, b1, W2, b2):
    grid_spec = pltpu.PrefetchScalarGridSpec(
        num_scalar_prefetch=1,
        grid=(NT, J2),
        in_specs=[
            pl.BlockSpec((BB, HD), lambda t, j, te: (t, 0)),
            pl.BlockSpec((1, HD, MLP), lambda t, j, te: (te[t], 0, 0)),
            pl.BlockSpec((E, MLP), lambda t, j, te: (0, 0)),
            pl.BlockSpec((1, MLPC, HD), lambda t, j, te: (te[t], j, 0)),
            pl.BlockSpec((E, HD), lambda t, j, te: (0, 0)),
        ],
        out_specs=pl.BlockSpec((BB, HD), lambda t, j, te: (t, 0)),
        scratch_shapes=[pltpu.VMEM((BB, MLP), jnp.float32)],
    )
    return pl.pallas_call(
        _gmm_body,
        grid_spec=grid_spec,
        out_shape=jax.ShapeDtypeStruct((NROWS, HD), jnp.float32),
        compiler_params=pltpu.CompilerParams(
            dimension_semantics=("arbitrary", "arbitrary")),
    )(te, xpad, W1, b1, W2, b2)


def _combine_body(y_hbm, pos0_hbm, pos1_hbm, c0_hbm, c1_hbm, out_hbm,
                  idx0_v, idx1_v, r0_v, r1_v, cc0_v, cc1_v, gsem, ssem):
    wid = lax.axis_index("s") * 2 + lax.axis_index("c")
    pltpu.sync_copy(pos0_hbm.at[wid], idx0_v)
    pltpu.sync_copy(pos1_hbm.at[wid], idx1_v)
    pltpu.sync_copy(c0_hbm.at[wid], cc0_v)
    pltpu.sync_copy(c1_hbm.at[wid], cc1_v)

    g0 = [None] * CNC
    g1 = [None] * CNC
    stores = [None] * CNC
    g0[0] = pltpu.async_copy(y_hbm.at[idx0_v.at[0]], r0_v.at[0], gsem)
    g1[0] = pltpu.async_copy(y_hbm.at[idx1_v.at[0]], r1_v.at[0], gsem)
    for ch in range(CNC):
        b = ch % 2
        g0[ch].wait()
        g1[ch].wait()
        if ch + 1 < CNC:
            if ch >= 1:
                stores[ch - 1].wait()
            g0[ch + 1] = pltpu.async_copy(
                y_hbm.at[idx0_v.at[ch + 1]], r0_v.at[1 - b], gsem)
            g1[ch + 1] = pltpu.async_copy(
                y_hbm.at[idx1_v.at[ch + 1]], r1_v.at[1 - b], gsem)

        r0b = r0_v.at[b]
        r1b = r1_v.at[b]
        cc0c = cc0_v.at[ch]
        cc1c = cc1_v.at[ch]

        def row(i, carry):
            a = cc0c[i, :]
            bb = cc1c[i, :]

            def col(k, carry2):
                sl = pl.ds(k * 16, 16)
                r0b[i, sl] = a * r0b[i, sl] + bb * r1b[i, sl]
                return carry2

            return lax.fori_loop(0, HD // 16, col, carry)

        lax.fori_loop(0, CCH, row, 0)
        base = wid * (CNC * CCH) + ch * CCH
        stores[ch] = pltpu.async_copy(r0_v.at[b], out_hbm.at[pl.ds(base, CCH)],
                                      ssem)
    stores[CNC - 2].wait()
    stores[CNC - 1].wait()


def _make_combine():
    return pl.kernel(
        _combine_body,
        out_type=jax.ShapeDtypeStruct((T, HD), jnp.float32),
        mesh=plsc.VectorSubcoreMesh(core_axis_name="c", subcore_axis_name="s"),
        scratch_types=[
            pltpu.VMEM((CNC, CCH), jnp.int32),
            pltpu.VMEM((CNC, CCH), jnp.int32),
            pltpu.VMEM((2, CCH, HD), jnp.float32),
            pltpu.VMEM((2, CCH, HD), jnp.float32),
            pltpu.VMEM((CNC, CCH, 16), jnp.float32),
            pltpu.VMEM((CNC, CCH, 16), jnp.float32),
            pltpu.SemaphoreType.DMA,
            pltpu.SemaphoreType.DMA,
        ],
    )


def kernel(inputs, Wr, br, W1, b1, W2, b2):
    flat = inputs.reshape(T, HD)
    gates, aux, pos0, pos1, c0, c1, te = _run_router(flat, Wr, br)
    xpad = _make_dispatch()(flat,
                            pos0.reshape(NW, DNC, DCH),
                            pos1.reshape(NW, DNC, DCH))
    y = _run_gmm(te.reshape(NT), xpad, W1, b1, W2, b2)
    out = _make_combine()(y,
                   pos0.reshape(NW, CNC, CCH),
                   pos1.reshape(NW, CNC, CCH),
                   c0.reshape(NW, CNC, CCH, 16),
                   c1.reshape(NW, CNC, CCH, 16))
    G = NS * (L // GS)
    return out.reshape(NS, L, HD), aux[0, 0], gates.reshape(G, GS, E)
